# baseline (speedup 1.0000x reference)
import jax
import jax.numpy as jnp
from jax import lax
from jax.experimental import pallas as pl

N_BINS = 1024
INPUT_DIM = 64
BN = 2048


def _vq_kernel(x_ref, x2_ref, v_ref, v2_ref, out_ref):
    xb = x_ref[...]
    v = v_ref[...]
    ab = lax.dot_general(
        xb, v, (((1,), (1,)), ((), ())),
        preferred_element_type=jnp.float32,
    )
    d2 = (x2_ref[...] + v2_ref[...]) - 2.0 * ab
    dist = jnp.sqrt(jnp.maximum(d2, 0.0))
    m = jnp.min(dist, axis=1, keepdims=True)
    iota = lax.broadcasted_iota(jnp.int32, dist.shape, 1)
    idx = jnp.min(jnp.where(dist == m, iota, N_BINS), axis=1, keepdims=True)
    out_ref[...] = idx


def kernel(x, vectors):
    shape = x.shape[:-1]
    flat = x.reshape(-1, x.shape[-1])
    n = flat.shape[0]
    x2 = jnp.sum(flat * flat, axis=-1, keepdims=True)
    v2 = jnp.sum(vectors * vectors, axis=-1)[None, :]

    grid = (n // BN,)
    out = pl.pallas_call(
        _vq_kernel,
        grid=grid,
        in_specs=[
            pl.BlockSpec((BN, INPUT_DIM), lambda i: (i, 0)),
            pl.BlockSpec((BN, 1), lambda i: (i, 0)),
            pl.BlockSpec((N_BINS, INPUT_DIM), lambda i: (0, 0)),
            pl.BlockSpec((1, N_BINS), lambda i: (0, 0)),
        ],
        out_specs=pl.BlockSpec((BN, 1), lambda i: (i, 0)),
        out_shape=jax.ShapeDtypeStruct((n, 1), jnp.int32),
    )(flat, x2, vectors, v2)
    return out.reshape(shape)
